# baseline (speedup 1.0000x reference)
import functools
import jax
import jax.numpy as jnp
from jax import lax
from jax.experimental import pallas as pl
from jax.experimental.pallas import tpu as pltpu

HID = 128
EDGE_BINS = 80
TRI_BINS = 40
EMB = 64
N_ALIGNN = 2
N_GCN = 2
BR = 512


def _sigmoid(x):
    return 1.0 / (1.0 + jnp.exp(-x))


def _stats(z, i, rows):
    rid = i * BR + lax.broadcasted_iota(jnp.int32, (BR, 1), 0)
    zm = jnp.where(rid < rows, z, 0.0)
    return jnp.sum(zm, axis=0), jnp.sum(zm * zm, axis=0)


def _mm_stats_body(x_ref, w_ref, b_ref, z_ref, p_ref, *, rows):
    i = pl.program_id(0)
    z = jnp.dot(x_ref[...], w_ref[...],
                preferred_element_type=jnp.float32) + b_ref[...]
    z_ref[...] = z
    s, q = _stats(z, i, rows)
    p_ref[0, 0, :] = s
    p_ref[0, 1, :] = q


def _linear_stats(x, W, b, rows):
    Rp, K = x.shape
    H = W.shape[1]
    nb = Rp // BR
    return pl.pallas_call(
        functools.partial(_mm_stats_body, rows=rows),
        grid=(nb,),
        in_specs=[pl.BlockSpec((BR, K), lambda i: (i, 0)),
                  pl.BlockSpec((K, H), lambda i: (0, 0)),
                  pl.BlockSpec((1, H), lambda i: (0, 0))],
        out_specs=[pl.BlockSpec((BR, H), lambda i: (i, 0)),
                   pl.BlockSpec((1, 8, H), lambda i: (i, 0, 0))],
        out_shape=[jax.ShapeDtypeStruct((Rp, H), jnp.float32),
                   jax.ShapeDtypeStruct((nb, 8, H), jnp.float32)],
    )(x, W, b.reshape(1, H))


def _bn_from_partials(p_ref, rows):
    s = jnp.sum(p_ref[:, 0, :], axis=0)
    q = jnp.sum(p_ref[:, 1, :], axis=0)
    mean = s / rows
    var = q / rows - mean * mean
    return mean, lax.rsqrt(var + 1e-5)


def _bn_silu_body(z_ref, p_ref, s_ref, b_ref, o_ref, *, rows):
    mean, inv = _bn_from_partials(p_ref, rows)
    t = (z_ref[...] - mean) * inv * s_ref[...] + b_ref[...]
    o_ref[...] = t * _sigmoid(t)


def _bn_silu_res_body(z_ref, p_ref, s_ref, b_ref, r_ref, o_ref, *, rows):
    mean, inv = _bn_from_partials(p_ref, rows)
    t = (z_ref[...] - mean) * inv * s_ref[...] + b_ref[...]
    o_ref[...] = r_ref[...] + t * _sigmoid(t)


def _bn_silu(z, p, s, b, rows, res=None):
    Rp, H = z.shape
    nb = Rp // BR
    pspec = pl.BlockSpec(p.shape, lambda i: (0, 0, 0))
    vspec = pl.BlockSpec((1, H), lambda i: (0, 0))
    bspec = pl.BlockSpec((BR, H), lambda i: (i, 0))
    if res is None:
        return pl.pallas_call(
            functools.partial(_bn_silu_body, rows=rows),
            grid=(nb,),
            in_specs=[bspec, pspec, vspec, vspec],
            out_specs=bspec,
            out_shape=jax.ShapeDtypeStruct((Rp, H), jnp.float32),
        )(z, p, s.reshape(1, H), b.reshape(1, H))
    return pl.pallas_call(
        functools.partial(_bn_silu_res_body, rows=rows),
        grid=(nb,),
        in_specs=[bspec, pspec, vspec, vspec, bspec],
        out_specs=bspec,
        out_shape=jax.ShapeDtypeStruct((Rp, H), jnp.float32),
    )(z, p, s.reshape(1, H), b.reshape(1, H), res)


def _nf_proj_body(x_ref, w_ref, b_ref, p1_ref, p2_ref, a3_ref):
    z = jnp.dot(x_ref[...], w_ref[...],
                preferred_element_type=jnp.float32) + b_ref[...]
    p1_ref[...] = z[:, 0:2 * HID]
    p2_ref[...] = z[:, 2 * HID:3 * HID]
    a3_ref[...] = z[:, 3 * HID:4 * HID]


def _nf_proj(x, Wcat, bcat):
    Rp = x.shape[0]
    nb = Rp // BR
    return pl.pallas_call(
        _nf_proj_body,
        grid=(nb,),
        in_specs=[pl.BlockSpec((BR, HID), lambda i: (i, 0)),
                  pl.BlockSpec((HID, 4 * HID), lambda i: (0, 0)),
                  pl.BlockSpec((1, 4 * HID), lambda i: (0, 0))],
        out_specs=[pl.BlockSpec((BR, 2 * HID), lambda i: (i, 0)),
                   pl.BlockSpec((BR, HID), lambda i: (i, 0)),
                   pl.BlockSpec((BR, HID), lambda i: (i, 0))],
        out_shape=[jax.ShapeDtypeStruct((Rp, 2 * HID), jnp.float32),
                   jax.ShapeDtypeStruct((Rp, HID), jnp.float32),
                   jax.ShapeDtypeStruct((Rp, HID), jnp.float32)],
    )(x, Wcat, bcat.reshape(1, 4 * HID))


def _msg_body(y_ref, g1_ref, g2_ref, w_ref, b_ref, m_ref, sg_ref, sh_ref,
              p_ref, *, rows):
    i = pl.program_id(0)
    m = (g1_ref[:, 0:HID] + g2_ref[...] +
         jnp.dot(y_ref[...], w_ref[...], preferred_element_type=jnp.float32)
         + b_ref[...])
    sg = _sigmoid(m)
    m_ref[...] = m
    sg_ref[...] = sg
    sh_ref[...] = g1_ref[:, HID:2 * HID] * sg
    s, q = _stats(m, i, rows)
    p_ref[0, 0, :] = s
    p_ref[0, 1, :] = q


def _msg(y, g1, g2, W2, b2, rows):
    Rp = y.shape[0]
    nb = Rp // BR
    bspec = pl.BlockSpec((BR, HID), lambda i: (i, 0))
    return pl.pallas_call(
        functools.partial(_msg_body, rows=rows),
        grid=(nb,),
        in_specs=[bspec,
                  pl.BlockSpec((BR, 2 * HID), lambda i: (i, 0)),
                  bspec,
                  pl.BlockSpec((HID, HID), lambda i: (0, 0)),
                  pl.BlockSpec((1, HID), lambda i: (0, 0))],
        out_specs=[bspec, bspec, bspec,
                   pl.BlockSpec((1, 8, HID), lambda i: (i, 0, 0))],
        out_shape=[jax.ShapeDtypeStruct((Rp, HID), jnp.float32),
                   jax.ShapeDtypeStruct((Rp, HID), jnp.float32),
                   jax.ShapeDtypeStruct((Rp, HID), jnp.float32),
                   jax.ShapeDtypeStruct((nb, 8, HID), jnp.float32)],
    )(y, g1, g2, W2, b2.reshape(1, HID))


def _gate_body(a3_ref, s1_ref, s2_ref, z_ref, p_ref, *, rows):
    i = pl.program_id(0)
    s1 = s1_ref[0] + s1_ref[1]
    s2 = s2_ref[0] + s2_ref[1]
    z = a3_ref[...] + s1 / (s2 + 1e-6)
    z_ref[...] = z
    s, q = _stats(z, i, rows)
    p_ref[0, 0, :] = s
    p_ref[0, 1, :] = q


def _gate(a3, s1, s2, rows):
    Rp = a3.shape[0]
    nb = Rp // BR
    bspec = pl.BlockSpec((BR, HID), lambda i: (i, 0))
    sspec = pl.BlockSpec((2, BR, HID), lambda i: (0, i, 0))
    return pl.pallas_call(
        functools.partial(_gate_body, rows=rows),
        grid=(nb,),
        in_specs=[bspec, sspec, sspec],
        out_specs=[bspec, pl.BlockSpec((1, 8, HID), lambda i: (i, 0, 0))],
        out_shape=[jax.ShapeDtypeStruct((Rp, HID), jnp.float32),
                   jax.ShapeDtypeStruct((nb, 8, HID), jnp.float32)],
    )(a3, s1, s2)


def _geo1_body(be_ref, ps_ref, pd_ref, tc_ref, cf_ref, er_ref, y0_ref):
    vec = pd_ref[...] - ps_ref[...] + tc_ref[...]
    bi = be_ref[0, 0, :]
    oh = (bi[:, None] == lax.broadcasted_iota(jnp.int32, (BR, 64), 1)
          ).astype(jnp.float32)
    ce = jnp.dot(oh, cf_ref[...], preferred_element_type=jnp.float32)
    v0, v1, v2 = vec[:, 0:1], vec[:, 1:2], vec[:, 2:3]
    er0 = ce[:, 0:1] * v0 + ce[:, 3:4] * v1 + ce[:, 6:7] * v2
    er1 = ce[:, 1:2] * v0 + ce[:, 4:5] * v1 + ce[:, 7:8] * v2
    er2 = ce[:, 2:3] * v0 + ce[:, 5:6] * v1 + ce[:, 8:9] * v2
    er_ref[...] = jnp.concatenate(
        [er0, er1, er2, jnp.zeros((BR, 13), jnp.float32)], axis=1)
    d = jnp.sqrt(er0 * er0 + er1 * er1 + er2 * er2)
    step = 8.0 / (EDGE_BINS - 1)
    centers = lax.broadcasted_iota(
        jnp.int32, (1, EDGE_BINS), 1).astype(jnp.float32) * step
    diff = d - centers
    y0_ref[...] = jnp.exp(-(1.0 / (step * step)) * diff * diff)


def _geo1(be3, ps, pd, tc16, cellflat):
    Rp = ps.shape[0]
    nb = Rp // BR
    v16 = pl.BlockSpec((BR, 16), lambda i: (i, 0))
    return pl.pallas_call(
        _geo1_body,
        grid=(nb,),
        in_specs=[pl.BlockSpec((1, 1, BR), lambda i: (i, 0, 0)),
                  v16, v16, v16,
                  pl.BlockSpec((64, 16), lambda i: (0, 0))],
        out_specs=[v16, pl.BlockSpec((BR, EDGE_BINS), lambda i: (i, 0))],
        out_shape=[jax.ShapeDtypeStruct((Rp, 16), jnp.float32),
                   jax.ShapeDtypeStruct((Rp, EDGE_BINS), jnp.float32)],
    )(be3, ps, pd, tc16, cellflat)


def _geo2_body(eu_ref, ev_ref, z0_ref):
    eu = eu_ref[...]
    ev = ev_ref[...]
    du = jnp.sum(eu * ev, axis=1, keepdims=True)
    nu = jnp.sum(eu * eu, axis=1, keepdims=True)
    nv = jnp.sum(ev * ev, axis=1, keepdims=True)
    cos = du / (jnp.sqrt(nu) * jnp.sqrt(nv))
    step = 2.0 / (TRI_BINS - 1)
    centers = -1.0 + lax.broadcasted_iota(
        jnp.int32, (1, TRI_BINS), 1).astype(jnp.float32) * step
    diff = cos - centers
    z0_ref[...] = jnp.exp(-(1.0 / (step * step)) * diff * diff)


def _geo2(eu, ev):
    Rp = eu.shape[0]
    nb = Rp // BR
    v16 = pl.BlockSpec((BR, 16), lambda i: (i, 0))
    return pl.pallas_call(
        _geo2_body,
        grid=(nb,),
        in_specs=[v16, v16],
        out_specs=pl.BlockSpec((BR, TRI_BINS), lambda i: (i, 0)),
        out_shape=jax.ShapeDtypeStruct((Rp, TRI_BINS), jnp.float32),
    )(eu, ev)


def _atom_body(z3_ref, zt_ref, w_ref, b_ref, o_ref, p_ref, *, rows):
    i = pl.program_id(0)
    zi = z3_ref[0, 0, :]
    oh = (zi[:, None] == lax.broadcasted_iota(jnp.int32, (BR, 100), 1)
          ).astype(jnp.float32)
    e = jnp.dot(oh, zt_ref[...], preferred_element_type=jnp.float32)
    z = jnp.dot(e, w_ref[...], preferred_element_type=jnp.float32) + b_ref[...]
    o_ref[...] = z
    s, q = _stats(z, i, rows)
    p_ref[0, 0, :] = s
    p_ref[0, 1, :] = q


def _atom_embed(z3, z_table, W, b, rows):
    Rp = z3.shape[0] * BR
    nb = z3.shape[0]
    return pl.pallas_call(
        functools.partial(_atom_body, rows=rows),
        grid=(nb,),
        in_specs=[pl.BlockSpec((1, 1, BR), lambda i: (i, 0, 0)),
                  pl.BlockSpec(z_table.shape, lambda i: (0, 0)),
                  pl.BlockSpec((z_table.shape[1], HID), lambda i: (0, 0)),
                  pl.BlockSpec((1, HID), lambda i: (0, 0))],
        out_specs=[pl.BlockSpec((BR, HID), lambda i: (i, 0)),
                   pl.BlockSpec((1, 8, HID), lambda i: (i, 0, 0))],
        out_shape=[jax.ShapeDtypeStruct((Rp, HID), jnp.float32),
                   jax.ShapeDtypeStruct((nb, 8, HID), jnp.float32)],
    )(z3, z_table, W, b.reshape(1, HID))


def _pool_body(x_ref, ba_ref, fw_ref, fb_ref, o_ref, h_ref, c_ref, *, rows,
               nseg, nb):
    i = pl.program_id(0)

    @pl.when(i == 0)
    def _():
        h_ref[...] = jnp.zeros_like(h_ref)
        c_ref[...] = jnp.zeros_like(c_ref)

    rid = i * BR + lax.broadcasted_iota(jnp.int32, (BR, 1), 0)
    bi = ba_ref[0, 0, :]
    oh = jnp.where(
        rid < rows,
        (bi[:, None] == lax.broadcasted_iota(jnp.int32, (BR, nseg), 1)
         ).astype(jnp.float32), 0.0)
    h_ref[...] += lax.dot_general(oh, x_ref[...], (((0,), (0,)), ((), ())),
                                  preferred_element_type=jnp.float32)
    c_ref[...] += jnp.sum(oh, axis=0)[:, None]

    @pl.when(i == nb - 1)
    def _():
        hm = h_ref[...] / jnp.maximum(c_ref[...], 1.0)
        o_ref[...] = jnp.dot(hm, fw_ref[...],
                             preferred_element_type=jnp.float32) + fb_ref[...]


def _pool(x, ba3, fc_W, fc_b, rows, nseg):
    Rp = x.shape[0]
    nb = Rp // BR
    return pl.pallas_call(
        functools.partial(_pool_body, rows=rows, nseg=nseg, nb=nb),
        grid=(nb,),
        in_specs=[pl.BlockSpec((BR, HID), lambda i: (i, 0)),
                  pl.BlockSpec((1, 1, BR), lambda i: (i, 0, 0)),
                  pl.BlockSpec((HID, 1), lambda i: (0, 0)),
                  pl.BlockSpec((1, 1), lambda i: (0, 0))],
        out_specs=pl.BlockSpec((nseg, 1), lambda i: (0, 0)),
        out_shape=jax.ShapeDtypeStruct((nseg, 1), jnp.float32),
        scratch_shapes=[pltpu.VMEM((nseg, HID), jnp.float32),
                        pltpu.VMEM((nseg, HID), jnp.float32)],
    )(x, ba3, fc_W, fc_b.reshape(1, 1))


def _gather_rows(table, idx):
    return table[idx]


def _segment_sum2(vals, idx, vpad):
    half = vals.shape[0] // 2
    s0 = jax.ops.segment_sum(vals[:half], idx[:half], num_segments=vpad)
    s1 = jax.ops.segment_sum(vals[half:], idx[half:], num_segments=vpad)
    return jnp.stack([s0, s1])


def _pad_rows(a, rp):
    pad = [(0, rp - a.shape[0])] + [(0, 0)] * (a.ndim - 1)
    return jnp.pad(a, pad)


def _eggc_pallas(srcp, dstp, dst_seg, nf, ef, W, b, bns, bnb, n_real, e_real,
                 vpad):
    Wcat = jnp.concatenate([W[0], W[4], W[1], W[3]], axis=1)
    bcat = jnp.concatenate([b[0], b[4], b[1], b[3]], axis=0)
    p1, p2, a3 = _nf_proj(nf, Wcat, bcat)
    g1 = _gather_rows(p1, srcp)
    g2 = _gather_rows(p2, dstp)
    m, sg, sh, pm = _msg(ef, g1, g2, W[2], b[2], e_real)
    s1 = _segment_sum2(sh, dst_seg, vpad)
    s2 = _segment_sum2(sg, dst_seg, vpad)
    xin, px = _gate(a3, s1, s2, n_real)
    x_new = _bn_silu(xin, px, bns[0], bnb[0], n_real, res=nf)
    y_new = _bn_silu(m, pm, bns[1], bnb[1], e_real, res=ef)
    return x_new, y_new


def kernel(edge_index, triplet_index, pos, cell, batch_edges, target_cell, z,
           batch_atoms, num_structures, z_table, atom_W, atom_b, atom_bn_s,
           atom_bn_b, edge_W1, edge_b1, edge_bn1_s, edge_bn1_b, edge_W2,
           edge_b2, edge_bn2_s, edge_bn2_b, ang_W1, ang_b1, ang_bn1_s,
           ang_bn1_b, ang_W2, ang_b2, ang_bn2_s, ang_bn2_b, eggc_W, eggc_b,
           eggc_bn_s, eggc_bn_b, fc_W, fc_b):
    n_nodes = pos.shape[0]
    n_edges = edge_index.shape[1]
    n_tri = triplet_index.shape[1]
    n_structures = cell.shape[0]
    npad = ((n_nodes + 2047) // 2048) * 2048
    epad = ((n_edges + 16383) // 16384) * 16384
    tpad = ((n_tri + 16383) // 16384) * 16384

    srcp = _pad_rows(edge_index[0].astype(jnp.int32), epad)
    dstp = _pad_rows(edge_index[1].astype(jnp.int32), epad)
    tup = _pad_rows(triplet_index[0].astype(jnp.int32), tpad)
    tvp = _pad_rows(triplet_index[1].astype(jnp.int32), tpad)
    dst_seg = jnp.where(
        jnp.arange(epad) < n_edges, dstp, jnp.int32(npad))
    tv_seg = jnp.where(
        jnp.arange(tpad) < n_tri, tvp, jnp.int32(epad))

    pos16 = _pad_rows(jnp.pad(pos, ((0, 0), (0, 13))), npad)
    tc16 = _pad_rows(jnp.pad(target_cell, ((0, 0), (0, 13))), epad)
    be3 = _pad_rows(batch_edges.astype(jnp.int32), epad).reshape(-1, 1, BR)
    ba3 = _pad_rows(batch_atoms.astype(jnp.int32), npad).reshape(-1, 1, BR)
    z3 = _pad_rows(z.astype(jnp.int32), npad).reshape(-1, 1, BR)
    cellflat = jnp.pad(cell.reshape(n_structures, 9), ((0, 0), (0, 7)))

    ps = _gather_rows(pos16, srcp)
    pd = _gather_rows(pos16, dstp)
    er16, y0 = _geo1(be3, ps, pd, tc16, cellflat)
    eu = _gather_rows(er16, tup)
    ev = _gather_rows(er16, tvp)
    z0 = _geo2(eu, ev)

    za, pa = _atom_embed(z3, z_table, atom_W, atom_b, n_nodes)
    x = _bn_silu(za, pa, atom_bn_s, atom_bn_b, n_nodes)
    z1, p1 = _linear_stats(y0, edge_W1, edge_b1, n_edges)
    y = _bn_silu(z1, p1, edge_bn1_s, edge_bn1_b, n_edges)
    z2, p2 = _linear_stats(y, edge_W2, edge_b2, n_edges)
    y = _bn_silu(z2, p2, edge_bn2_s, edge_bn2_b, n_edges)
    z3f, p3 = _linear_stats(z0, ang_W1, ang_b1, n_tri)
    zf = _bn_silu(z3f, p3, ang_bn1_s, ang_bn1_b, n_tri)
    z4, p4 = _linear_stats(zf, ang_W2, ang_b2, n_tri)
    zf = _bn_silu(z4, p4, ang_bn2_s, ang_bn2_b, n_tri)

    for i in range(N_ALIGNN):
        x, m = _eggc_pallas(srcp, dstp, dst_seg, x, y, eggc_W[2 * i],
                            eggc_b[2 * i], eggc_bn_s[2 * i], eggc_bn_b[2 * i],
                            n_nodes, n_edges, npad)
        y, zf = _eggc_pallas(tup, tvp, tv_seg, m, zf, eggc_W[2 * i + 1],
                             eggc_b[2 * i + 1], eggc_bn_s[2 * i + 1],
                             eggc_bn_b[2 * i + 1], n_edges, n_tri, epad)
    for i in range(N_GCN):
        k = 2 * N_ALIGNN + i
        x, y = _eggc_pallas(srcp, dstp, dst_seg, x, y, eggc_W[k], eggc_b[k],
                            eggc_bn_s[k], eggc_bn_b[k], n_nodes, n_edges,
                            npad)

    out = _pool(x, ba3, fc_W, fc_b, n_nodes, n_structures)
    return jnp.squeeze(out)


# speedup vs baseline: 1.0642x; 1.0642x over previous
import functools
import jax
import jax.numpy as jnp
from jax import lax
from jax.experimental import pallas as pl
from jax.experimental.pallas import tpu as pltpu
from jax.experimental.pallas import tpu_sc as plsc

HID = 128
EDGE_BINS = 80
TRI_BINS = 40
EMB = 64
N_ALIGNN = 2
N_GCN = 2
BR = 512


def _sigmoid(x):
    return 1.0 / (1.0 + jnp.exp(-x))


def _stats(z, i, rows):
    rid = i * BR + lax.broadcasted_iota(jnp.int32, (BR, 1), 0)
    zm = jnp.where(rid < rows, z, 0.0)
    return jnp.sum(zm, axis=0), jnp.sum(zm * zm, axis=0)


def _mm_stats_body(x_ref, w_ref, b_ref, z_ref, p_ref, *, rows):
    i = pl.program_id(0)
    z = jnp.dot(x_ref[...], w_ref[...],
                preferred_element_type=jnp.float32) + b_ref[...]
    z_ref[...] = z
    s, q = _stats(z, i, rows)
    p_ref[0, 0, :] = s
    p_ref[0, 1, :] = q


def _linear_stats(x, W, b, rows):
    Rp, K = x.shape
    H = W.shape[1]
    nb = Rp // BR
    return pl.pallas_call(
        functools.partial(_mm_stats_body, rows=rows),
        grid=(nb,),
        in_specs=[pl.BlockSpec((BR, K), lambda i: (i, 0)),
                  pl.BlockSpec((K, H), lambda i: (0, 0)),
                  pl.BlockSpec((1, H), lambda i: (0, 0))],
        out_specs=[pl.BlockSpec((BR, H), lambda i: (i, 0)),
                   pl.BlockSpec((1, 8, H), lambda i: (i, 0, 0))],
        out_shape=[jax.ShapeDtypeStruct((Rp, H), jnp.float32),
                   jax.ShapeDtypeStruct((nb, 8, H), jnp.float32)],
    )(x, W, b.reshape(1, H))


def _bn_from_partials(p_ref, rows):
    s = jnp.sum(p_ref[:, 0, :], axis=0)
    q = jnp.sum(p_ref[:, 1, :], axis=0)
    mean = s / rows
    var = q / rows - mean * mean
    return mean, lax.rsqrt(var + 1e-5)


def _bn_silu_body(z_ref, p_ref, s_ref, b_ref, o_ref, *, rows):
    mean, inv = _bn_from_partials(p_ref, rows)
    t = (z_ref[...] - mean) * inv * s_ref[...] + b_ref[...]
    o_ref[...] = t * _sigmoid(t)


def _bn_silu_res_body(z_ref, p_ref, s_ref, b_ref, r_ref, o_ref, *, rows):
    mean, inv = _bn_from_partials(p_ref, rows)
    t = (z_ref[...] - mean) * inv * s_ref[...] + b_ref[...]
    o_ref[...] = r_ref[...] + t * _sigmoid(t)


def _bn_silu(z, p, s, b, rows, res=None):
    Rp, H = z.shape
    nb = Rp // BR
    pspec = pl.BlockSpec(p.shape, lambda i: (0, 0, 0))
    vspec = pl.BlockSpec((1, H), lambda i: (0, 0))
    bspec = pl.BlockSpec((BR, H), lambda i: (i, 0))
    if res is None:
        return pl.pallas_call(
            functools.partial(_bn_silu_body, rows=rows),
            grid=(nb,),
            in_specs=[bspec, pspec, vspec, vspec],
            out_specs=bspec,
            out_shape=jax.ShapeDtypeStruct((Rp, H), jnp.float32),
        )(z, p, s.reshape(1, H), b.reshape(1, H))
    return pl.pallas_call(
        functools.partial(_bn_silu_res_body, rows=rows),
        grid=(nb,),
        in_specs=[bspec, pspec, vspec, vspec, bspec],
        out_specs=bspec,
        out_shape=jax.ShapeDtypeStruct((Rp, H), jnp.float32),
    )(z, p, s.reshape(1, H), b.reshape(1, H), res)


def _nf_proj_body(x_ref, w_ref, b_ref, p1_ref, p2_ref, a3_ref):
    z = jnp.dot(x_ref[...], w_ref[...],
                preferred_element_type=jnp.float32) + b_ref[...]
    p1_ref[...] = z[:, 0:2 * HID]
    p2_ref[...] = z[:, 2 * HID:3 * HID]
    a3_ref[...] = z[:, 3 * HID:4 * HID]


def _nf_proj(x, Wcat, bcat):
    Rp = x.shape[0]
    nb = Rp // BR
    return pl.pallas_call(
        _nf_proj_body,
        grid=(nb,),
        in_specs=[pl.BlockSpec((BR, HID), lambda i: (i, 0)),
                  pl.BlockSpec((HID, 4 * HID), lambda i: (0, 0)),
                  pl.BlockSpec((1, 4 * HID), lambda i: (0, 0))],
        out_specs=[pl.BlockSpec((BR, 2 * HID), lambda i: (i, 0)),
                   pl.BlockSpec((BR, HID), lambda i: (i, 0)),
                   pl.BlockSpec((BR, HID), lambda i: (i, 0))],
        out_shape=[jax.ShapeDtypeStruct((Rp, 2 * HID), jnp.float32),
                   jax.ShapeDtypeStruct((Rp, HID), jnp.float32),
                   jax.ShapeDtypeStruct((Rp, HID), jnp.float32)],
    )(x, Wcat, bcat.reshape(1, 4 * HID))


def _msg_body(y_ref, g1_ref, g2_ref, w_ref, b_ref, m_ref, sg_ref, sh_ref,
              p_ref, *, rows):
    i = pl.program_id(0)
    m = (g1_ref[:, 0:HID] + g2_ref[...] +
         jnp.dot(y_ref[...], w_ref[...], preferred_element_type=jnp.float32)
         + b_ref[...])
    sg = _sigmoid(m)
    m_ref[...] = m
    sg_ref[...] = sg
    sh_ref[...] = g1_ref[:, HID:2 * HID] * sg
    s, q = _stats(m, i, rows)
    p_ref[0, 0, :] = s
    p_ref[0, 1, :] = q


def _msg(y, g1, g2, W2, b2, rows):
    Rp = y.shape[0]
    nb = Rp // BR
    bspec = pl.BlockSpec((BR, HID), lambda i: (i, 0))
    return pl.pallas_call(
        functools.partial(_msg_body, rows=rows),
        grid=(nb,),
        in_specs=[bspec,
                  pl.BlockSpec((BR, 2 * HID), lambda i: (i, 0)),
                  bspec,
                  pl.BlockSpec((HID, HID), lambda i: (0, 0)),
                  pl.BlockSpec((1, HID), lambda i: (0, 0))],
        out_specs=[bspec, bspec, bspec,
                   pl.BlockSpec((1, 8, HID), lambda i: (i, 0, 0))],
        out_shape=[jax.ShapeDtypeStruct((Rp, HID), jnp.float32),
                   jax.ShapeDtypeStruct((Rp, HID), jnp.float32),
                   jax.ShapeDtypeStruct((Rp, HID), jnp.float32),
                   jax.ShapeDtypeStruct((nb, 8, HID), jnp.float32)],
    )(y, g1, g2, W2, b2.reshape(1, HID))


def _gate_body(a3_ref, s1_ref, s2_ref, z_ref, p_ref, *, rows):
    i = pl.program_id(0)
    s1 = s1_ref[0] + s1_ref[1]
    s2 = s2_ref[0] + s2_ref[1]
    z = a3_ref[...] + s1 / (s2 + 1e-6)
    z_ref[...] = z
    s, q = _stats(z, i, rows)
    p_ref[0, 0, :] = s
    p_ref[0, 1, :] = q


def _gate(a3, s1, s2, rows):
    Rp = a3.shape[0]
    nb = Rp // BR
    bspec = pl.BlockSpec((BR, HID), lambda i: (i, 0))
    sspec = pl.BlockSpec((2, BR, HID), lambda i: (0, i, 0))
    return pl.pallas_call(
        functools.partial(_gate_body, rows=rows),
        grid=(nb,),
        in_specs=[bspec, sspec, sspec],
        out_specs=[bspec, pl.BlockSpec((1, 8, HID), lambda i: (i, 0, 0))],
        out_shape=[jax.ShapeDtypeStruct((Rp, HID), jnp.float32),
                   jax.ShapeDtypeStruct((nb, 8, HID), jnp.float32)],
    )(a3, s1, s2)


def _geo1_body(be_ref, ps_ref, pd_ref, tc_ref, cf_ref, er_ref, y0_ref):
    vec = pd_ref[...] - ps_ref[...] + tc_ref[...]
    bi = be_ref[0, 0, :]
    oh = (bi[:, None] == lax.broadcasted_iota(jnp.int32, (BR, 64), 1)
          ).astype(jnp.float32)
    ce = jnp.dot(oh, cf_ref[...], preferred_element_type=jnp.float32)
    v0, v1, v2 = vec[:, 0:1], vec[:, 1:2], vec[:, 2:3]
    er0 = ce[:, 0:1] * v0 + ce[:, 3:4] * v1 + ce[:, 6:7] * v2
    er1 = ce[:, 1:2] * v0 + ce[:, 4:5] * v1 + ce[:, 7:8] * v2
    er2 = ce[:, 2:3] * v0 + ce[:, 5:6] * v1 + ce[:, 8:9] * v2
    er_ref[...] = jnp.concatenate(
        [er0, er1, er2, jnp.zeros((BR, 125), jnp.float32)], axis=1)
    d = jnp.sqrt(er0 * er0 + er1 * er1 + er2 * er2)
    step = 8.0 / (EDGE_BINS - 1)
    centers = lax.broadcasted_iota(
        jnp.int32, (1, EDGE_BINS), 1).astype(jnp.float32) * step
    diff = d - centers
    y0_ref[...] = jnp.exp(-(1.0 / (step * step)) * diff * diff)


def _geo1(be3, ps, pd, tc16, cellflat):
    Rp = ps.shape[0]
    nb = Rp // BR
    v16 = pl.BlockSpec((BR, 128), lambda i: (i, 0))
    return pl.pallas_call(
        _geo1_body,
        grid=(nb,),
        in_specs=[pl.BlockSpec((1, 1, BR), lambda i: (i, 0, 0)),
                  v16, v16, v16,
                  pl.BlockSpec((64, 16), lambda i: (0, 0))],
        out_specs=[v16, pl.BlockSpec((BR, EDGE_BINS), lambda i: (i, 0))],
        out_shape=[jax.ShapeDtypeStruct((Rp, 128), jnp.float32),
                   jax.ShapeDtypeStruct((Rp, EDGE_BINS), jnp.float32)],
    )(be3, ps, pd, tc16, cellflat)


def _geo2_body(eu_ref, ev_ref, z0_ref):
    eu = eu_ref[...]
    ev = ev_ref[...]
    du = jnp.sum(eu * ev, axis=1, keepdims=True)
    nu = jnp.sum(eu * eu, axis=1, keepdims=True)
    nv = jnp.sum(ev * ev, axis=1, keepdims=True)
    cos = du / (jnp.sqrt(nu) * jnp.sqrt(nv))
    step = 2.0 / (TRI_BINS - 1)
    centers = -1.0 + lax.broadcasted_iota(
        jnp.int32, (1, TRI_BINS), 1).astype(jnp.float32) * step
    diff = cos - centers
    z0_ref[...] = jnp.exp(-(1.0 / (step * step)) * diff * diff)


def _geo2(eu, ev):
    Rp = eu.shape[0]
    nb = Rp // BR
    v16 = pl.BlockSpec((BR, 128), lambda i: (i, 0))
    return pl.pallas_call(
        _geo2_body,
        grid=(nb,),
        in_specs=[v16, v16],
        out_specs=pl.BlockSpec((BR, TRI_BINS), lambda i: (i, 0)),
        out_shape=jax.ShapeDtypeStruct((Rp, TRI_BINS), jnp.float32),
    )(eu, ev)


def _atom_body(z3_ref, zt_ref, w_ref, b_ref, o_ref, p_ref, *, rows):
    i = pl.program_id(0)
    zi = z3_ref[0, 0, :]
    oh = (zi[:, None] == lax.broadcasted_iota(jnp.int32, (BR, 100), 1)
          ).astype(jnp.float32)
    e = jnp.dot(oh, zt_ref[...], preferred_element_type=jnp.float32)
    z = jnp.dot(e, w_ref[...], preferred_element_type=jnp.float32) + b_ref[...]
    o_ref[...] = z
    s, q = _stats(z, i, rows)
    p_ref[0, 0, :] = s
    p_ref[0, 1, :] = q


def _atom_embed(z3, z_table, W, b, rows):
    Rp = z3.shape[0] * BR
    nb = z3.shape[0]
    return pl.pallas_call(
        functools.partial(_atom_body, rows=rows),
        grid=(nb,),
        in_specs=[pl.BlockSpec((1, 1, BR), lambda i: (i, 0, 0)),
                  pl.BlockSpec(z_table.shape, lambda i: (0, 0)),
                  pl.BlockSpec((z_table.shape[1], HID), lambda i: (0, 0)),
                  pl.BlockSpec((1, HID), lambda i: (0, 0))],
        out_specs=[pl.BlockSpec((BR, HID), lambda i: (i, 0)),
                   pl.BlockSpec((1, 8, HID), lambda i: (i, 0, 0))],
        out_shape=[jax.ShapeDtypeStruct((Rp, HID), jnp.float32),
                   jax.ShapeDtypeStruct((nb, 8, HID), jnp.float32)],
    )(z3, z_table, W, b.reshape(1, HID))


def _pool_body(x_ref, ba_ref, fw_ref, fb_ref, o_ref, h_ref, c_ref, *, rows,
               nseg, nb):
    i = pl.program_id(0)

    @pl.when(i == 0)
    def _():
        h_ref[...] = jnp.zeros_like(h_ref)
        c_ref[...] = jnp.zeros_like(c_ref)

    rid = i * BR + lax.broadcasted_iota(jnp.int32, (BR, 1), 0)
    bi = ba_ref[0, 0, :]
    oh = jnp.where(
        rid < rows,
        (bi[:, None] == lax.broadcasted_iota(jnp.int32, (BR, nseg), 1)
         ).astype(jnp.float32), 0.0)
    h_ref[...] += lax.dot_general(oh, x_ref[...], (((0,), (0,)), ((), ())),
                                  preferred_element_type=jnp.float32)
    c_ref[...] += jnp.sum(oh, axis=0)[:, None]

    @pl.when(i == nb - 1)
    def _():
        hm = h_ref[...] / jnp.maximum(c_ref[...], 1.0)
        o_ref[...] = jnp.dot(hm, fw_ref[...],
                             preferred_element_type=jnp.float32) + fb_ref[...]


def _pool(x, ba3, fc_W, fc_b, rows, nseg):
    Rp = x.shape[0]
    nb = Rp // BR
    return pl.pallas_call(
        functools.partial(_pool_body, rows=rows, nseg=nseg, nb=nb),
        grid=(nb,),
        in_specs=[pl.BlockSpec((BR, HID), lambda i: (i, 0)),
                  pl.BlockSpec((1, 1, BR), lambda i: (i, 0, 0)),
                  pl.BlockSpec((HID, 1), lambda i: (0, 0)),
                  pl.BlockSpec((1, 1), lambda i: (0, 0))],
        out_specs=pl.BlockSpec((nseg, 1), lambda i: (0, 0)),
        out_shape=jax.ShapeDtypeStruct((nseg, 1), jnp.float32),
        scratch_shapes=[pltpu.VMEM((nseg, HID), jnp.float32),
                        pltpu.VMEM((nseg, HID), jnp.float32)],
    )(x, ba3, fc_W, fc_b.reshape(1, 1))


def _sc_info():
    info = plsc.get_sparse_core_info()
    return info.num_cores, info.num_subcores


def _gather_rows(table, idx):
    V, D = table.shape
    B = idx.shape[0]
    nc, ns = _sc_info()
    nw = nc * ns
    bw = B // nw
    ch = min(1280, 40960 // D)
    nch = bw // ch
    mesh = plsc.VectorSubcoreMesh(core_axis_name="c", subcore_axis_name="s")

    @functools.partial(
        pl.kernel, mesh=mesh,
        out_type=jax.ShapeDtypeStruct((B, D), jnp.float32),
        scratch_types=[pltpu.VMEM((ch,), jnp.int32),
                       pltpu.VMEM((ch, D), jnp.float32),
                       pltpu.SemaphoreType.DMA],
    )
    def k(table_hbm, idx_hbm, out_hbm, idx_v, rows_v, sem):
        wid = lax.axis_index("s") * nc + lax.axis_index("c")
        base = wid * bw

        def body(j, carry):
            off = base + j * ch
            pltpu.sync_copy(idx_hbm.at[pl.ds(off, ch)], idx_v)
            pltpu.async_copy(table_hbm.at[idx_v], rows_v, sem).wait()
            pltpu.sync_copy(rows_v, out_hbm.at[pl.ds(off, ch)])
            return carry

        lax.fori_loop(0, nch, body, 0)

    return k(table, idx)


def _segment_sum2(vals, idx, vpad):
    half = vals.shape[0] // 2
    s0 = jax.ops.segment_sum(vals[:half], idx[:half], num_segments=vpad)
    s1 = jax.ops.segment_sum(vals[half:], idx[half:], num_segments=vpad)
    return jnp.stack([s0, s1])


def _pad_rows(a, rp):
    pad = [(0, rp - a.shape[0])] + [(0, 0)] * (a.ndim - 1)
    return jnp.pad(a, pad)


def _eggc_pallas(srcp, dstp, dst_seg, nf, ef, W, b, bns, bnb, n_real, e_real,
                 vpad):
    Wcat = jnp.concatenate([W[0], W[4], W[1], W[3]], axis=1)
    bcat = jnp.concatenate([b[0], b[4], b[1], b[3]], axis=0)
    p1, p2, a3 = _nf_proj(nf, Wcat, bcat)
    g1 = _gather_rows(p1, srcp)
    g2 = _gather_rows(p2, dstp)
    m, sg, sh, pm = _msg(ef, g1, g2, W[2], b[2], e_real)
    s1 = _segment_sum2(sh, dst_seg, vpad)
    s2 = _segment_sum2(sg, dst_seg, vpad)
    xin, px = _gate(a3, s1, s2, n_real)
    x_new = _bn_silu(xin, px, bns[0], bnb[0], n_real, res=nf)
    y_new = _bn_silu(m, pm, bns[1], bnb[1], e_real, res=ef)
    return x_new, y_new


def kernel(edge_index, triplet_index, pos, cell, batch_edges, target_cell, z,
           batch_atoms, num_structures, z_table, atom_W, atom_b, atom_bn_s,
           atom_bn_b, edge_W1, edge_b1, edge_bn1_s, edge_bn1_b, edge_W2,
           edge_b2, edge_bn2_s, edge_bn2_b, ang_W1, ang_b1, ang_bn1_s,
           ang_bn1_b, ang_W2, ang_b2, ang_bn2_s, ang_bn2_b, eggc_W, eggc_b,
           eggc_bn_s, eggc_bn_b, fc_W, fc_b):
    n_nodes = pos.shape[0]
    n_edges = edge_index.shape[1]
    n_tri = triplet_index.shape[1]
    n_structures = cell.shape[0]
    npad = ((n_nodes + 2047) // 2048) * 2048
    epad = ((n_edges + 16383) // 16384) * 16384
    tpad = ((n_tri + 16383) // 16384) * 16384

    srcp = _pad_rows(edge_index[0].astype(jnp.int32), epad)
    dstp = _pad_rows(edge_index[1].astype(jnp.int32), epad)
    tup = _pad_rows(triplet_index[0].astype(jnp.int32), tpad)
    tvp = _pad_rows(triplet_index[1].astype(jnp.int32), tpad)
    dst_seg = jnp.where(
        jnp.arange(epad) < n_edges, dstp, jnp.int32(npad))
    tv_seg = jnp.where(
        jnp.arange(tpad) < n_tri, tvp, jnp.int32(epad))

    pos16 = _pad_rows(jnp.pad(pos, ((0, 0), (0, 125))), npad)
    tc16 = _pad_rows(jnp.pad(target_cell, ((0, 0), (0, 125))), epad)
    be3 = _pad_rows(batch_edges.astype(jnp.int32), epad).reshape(-1, 1, BR)
    ba3 = _pad_rows(batch_atoms.astype(jnp.int32), npad).reshape(-1, 1, BR)
    z3 = _pad_rows(z.astype(jnp.int32), npad).reshape(-1, 1, BR)
    cellflat = jnp.pad(cell.reshape(n_structures, 9), ((0, 0), (0, 7)))

    ps = _gather_rows(pos16, srcp)
    pd = _gather_rows(pos16, dstp)
    er16, y0 = _geo1(be3, ps, pd, tc16, cellflat)
    eu = _gather_rows(er16, tup)
    ev = _gather_rows(er16, tvp)
    z0 = _geo2(eu, ev)

    za, pa = _atom_embed(z3, z_table, atom_W, atom_b, n_nodes)
    x = _bn_silu(za, pa, atom_bn_s, atom_bn_b, n_nodes)
    z1, p1 = _linear_stats(y0, edge_W1, edge_b1, n_edges)
    y = _bn_silu(z1, p1, edge_bn1_s, edge_bn1_b, n_edges)
    z2, p2 = _linear_stats(y, edge_W2, edge_b2, n_edges)
    y = _bn_silu(z2, p2, edge_bn2_s, edge_bn2_b, n_edges)
    z3f, p3 = _linear_stats(z0, ang_W1, ang_b1, n_tri)
    zf = _bn_silu(z3f, p3, ang_bn1_s, ang_bn1_b, n_tri)
    z4, p4 = _linear_stats(zf, ang_W2, ang_b2, n_tri)
    zf = _bn_silu(z4, p4, ang_bn2_s, ang_bn2_b, n_tri)

    for i in range(N_ALIGNN):
        x, m = _eggc_pallas(srcp, dstp, dst_seg, x, y, eggc_W[2 * i],
                            eggc_b[2 * i], eggc_bn_s[2 * i], eggc_bn_b[2 * i],
                            n_nodes, n_edges, npad)
        y, zf = _eggc_pallas(tup, tvp, tv_seg, m, zf, eggc_W[2 * i + 1],
                             eggc_b[2 * i + 1], eggc_bn_s[2 * i + 1],
                             eggc_bn_b[2 * i + 1], n_edges, n_tri, epad)
    for i in range(N_GCN):
        k = 2 * N_ALIGNN + i
        x, y = _eggc_pallas(srcp, dstp, dst_seg, x, y, eggc_W[k], eggc_b[k],
                            eggc_bn_s[k], eggc_bn_b[k], n_nodes, n_edges,
                            npad)

    out = _pool(x, ba3, fc_W, fc_b, n_nodes, n_structures)
    return jnp.squeeze(out)


# speedup vs baseline: 1.0784x; 1.0133x over previous
import functools
import jax
import jax.numpy as jnp
from jax import lax
from jax.experimental import pallas as pl
from jax.experimental.pallas import tpu as pltpu
from jax.experimental.pallas import tpu_sc as plsc

HID = 128
EDGE_BINS = 80
TRI_BINS = 40
EMB = 64
N_ALIGNN = 2
N_GCN = 2
BR = 512


def _sigmoid(x):
    return 1.0 / (1.0 + jnp.exp(-x))


def _stats(z, i, rows):
    rid = i * BR + lax.broadcasted_iota(jnp.int32, (BR, 1), 0)
    zm = jnp.where(rid < rows, z, 0.0)
    return jnp.sum(zm, axis=0), jnp.sum(zm * zm, axis=0)


def _mm_stats_body(x_ref, w_ref, b_ref, z_ref, p_ref, *, rows):
    i = pl.program_id(0)
    z = jnp.dot(x_ref[...], w_ref[...],
                preferred_element_type=jnp.float32) + b_ref[...]
    z_ref[...] = z
    s, q = _stats(z, i, rows)
    p_ref[0, 0, :] = s
    p_ref[0, 1, :] = q


def _linear_stats(x, W, b, rows):
    Rp, K = x.shape
    H = W.shape[1]
    nb = Rp // BR
    return pl.pallas_call(
        functools.partial(_mm_stats_body, rows=rows),
        grid=(nb,),
        in_specs=[pl.BlockSpec((BR, K), lambda i: (i, 0)),
                  pl.BlockSpec((K, H), lambda i: (0, 0)),
                  pl.BlockSpec((1, H), lambda i: (0, 0))],
        out_specs=[pl.BlockSpec((BR, H), lambda i: (i, 0)),
                   pl.BlockSpec((1, 8, H), lambda i: (i, 0, 0))],
        out_shape=[jax.ShapeDtypeStruct((Rp, H), jnp.float32),
                   jax.ShapeDtypeStruct((nb, 8, H), jnp.float32)],
    )(x, W, b.reshape(1, H))


def _bn_from_partials(p_ref, rows):
    s = jnp.sum(p_ref[:, 0, :], axis=0)
    q = jnp.sum(p_ref[:, 1, :], axis=0)
    mean = s / rows
    var = q / rows - mean * mean
    return mean, lax.rsqrt(var + 1e-5)


def _bn_silu_body(z_ref, p_ref, s_ref, b_ref, o_ref, *, rows):
    mean, inv = _bn_from_partials(p_ref, rows)
    t = (z_ref[...] - mean) * inv * s_ref[...] + b_ref[...]
    o_ref[...] = t * _sigmoid(t)


def _bn_silu_res_body(z_ref, p_ref, s_ref, b_ref, r_ref, o_ref, *, rows):
    mean, inv = _bn_from_partials(p_ref, rows)
    t = (z_ref[...] - mean) * inv * s_ref[...] + b_ref[...]
    o_ref[...] = r_ref[...] + t * _sigmoid(t)


def _bn_silu(z, p, s, b, rows, res=None):
    Rp, H = z.shape
    nb = Rp // BR
    pspec = pl.BlockSpec(p.shape, lambda i: (0, 0, 0))
    vspec = pl.BlockSpec((1, H), lambda i: (0, 0))
    bspec = pl.BlockSpec((BR, H), lambda i: (i, 0))
    if res is None:
        return pl.pallas_call(
            functools.partial(_bn_silu_body, rows=rows),
            grid=(nb,),
            in_specs=[bspec, pspec, vspec, vspec],
            out_specs=bspec,
            out_shape=jax.ShapeDtypeStruct((Rp, H), jnp.float32),
        )(z, p, s.reshape(1, H), b.reshape(1, H))
    return pl.pallas_call(
        functools.partial(_bn_silu_res_body, rows=rows),
        grid=(nb,),
        in_specs=[bspec, pspec, vspec, vspec, bspec],
        out_specs=bspec,
        out_shape=jax.ShapeDtypeStruct((Rp, H), jnp.float32),
    )(z, p, s.reshape(1, H), b.reshape(1, H), res)


def _nf_proj_body(x_ref, w_ref, b_ref, p1_ref, p2_ref, a3_ref):
    z = jnp.dot(x_ref[...], w_ref[...],
                preferred_element_type=jnp.float32) + b_ref[...]
    p1_ref[...] = z[:, 0:2 * HID]
    p2_ref[...] = z[:, 2 * HID:3 * HID]
    a3_ref[...] = z[:, 3 * HID:4 * HID]


def _nf_proj(x, Wcat, bcat):
    Rp = x.shape[0]
    nb = Rp // BR
    return pl.pallas_call(
        _nf_proj_body,
        grid=(nb,),
        in_specs=[pl.BlockSpec((BR, HID), lambda i: (i, 0)),
                  pl.BlockSpec((HID, 4 * HID), lambda i: (0, 0)),
                  pl.BlockSpec((1, 4 * HID), lambda i: (0, 0))],
        out_specs=[pl.BlockSpec((BR, 2 * HID), lambda i: (i, 0)),
                   pl.BlockSpec((BR, HID), lambda i: (i, 0)),
                   pl.BlockSpec((BR, HID), lambda i: (i, 0))],
        out_shape=[jax.ShapeDtypeStruct((Rp, 2 * HID), jnp.float32),
                   jax.ShapeDtypeStruct((Rp, HID), jnp.float32),
                   jax.ShapeDtypeStruct((Rp, HID), jnp.float32)],
    )(x, Wcat, bcat.reshape(1, 4 * HID))


def _msg_body(y_ref, g1_ref, g2_ref, w_ref, b_ref, m_ref, sg_ref, sh_ref,
              p_ref, *, rows):
    i = pl.program_id(0)
    m = (g1_ref[:, 0:HID] + g2_ref[...] +
         jnp.dot(y_ref[...], w_ref[...], preferred_element_type=jnp.float32)
         + b_ref[...])
    sg = _sigmoid(m)
    m_ref[...] = m
    sg_ref[...] = sg
    sh_ref[...] = g1_ref[:, HID:2 * HID] * sg
    s, q = _stats(m, i, rows)
    p_ref[0, 0, :] = s
    p_ref[0, 1, :] = q


def _msg(y, g1, g2, W2, b2, rows):
    Rp = y.shape[0]
    nb = Rp // BR
    bspec = pl.BlockSpec((BR, HID), lambda i: (i, 0))
    return pl.pallas_call(
        functools.partial(_msg_body, rows=rows),
        grid=(nb,),
        in_specs=[bspec,
                  pl.BlockSpec((BR, 2 * HID), lambda i: (i, 0)),
                  bspec,
                  pl.BlockSpec((HID, HID), lambda i: (0, 0)),
                  pl.BlockSpec((1, HID), lambda i: (0, 0))],
        out_specs=[bspec, bspec, bspec,
                   pl.BlockSpec((1, 8, HID), lambda i: (i, 0, 0))],
        out_shape=[jax.ShapeDtypeStruct((Rp, HID), jnp.float32),
                   jax.ShapeDtypeStruct((Rp, HID), jnp.float32),
                   jax.ShapeDtypeStruct((Rp, HID), jnp.float32),
                   jax.ShapeDtypeStruct((nb, 8, HID), jnp.float32)],
    )(y, g1, g2, W2, b2.reshape(1, HID))


def _gate_body(a3_ref, s1_ref, s2_ref, z_ref, p_ref, *, rows):
    i = pl.program_id(0)
    s1 = s1_ref[0] + s1_ref[1]
    s2 = s2_ref[0] + s2_ref[1]
    z = a3_ref[...] + s1 / (s2 + 1e-6)
    z_ref[...] = z
    s, q = _stats(z, i, rows)
    p_ref[0, 0, :] = s
    p_ref[0, 1, :] = q


def _gate(a3, s1, s2, rows):
    Rp = a3.shape[0]
    nb = Rp // BR
    bspec = pl.BlockSpec((BR, HID), lambda i: (i, 0))
    sspec = pl.BlockSpec((2, BR, HID), lambda i: (0, i, 0))
    return pl.pallas_call(
        functools.partial(_gate_body, rows=rows),
        grid=(nb,),
        in_specs=[bspec, sspec, sspec],
        out_specs=[bspec, pl.BlockSpec((1, 8, HID), lambda i: (i, 0, 0))],
        out_shape=[jax.ShapeDtypeStruct((Rp, HID), jnp.float32),
                   jax.ShapeDtypeStruct((nb, 8, HID), jnp.float32)],
    )(a3, s1, s2)


def _geo1_body(be_ref, ps_ref, pd_ref, tc_ref, cf_ref, er_ref, y0_ref):
    vec = pd_ref[...] - ps_ref[...] + tc_ref[...]
    bi = be_ref[0, 0, :]
    oh = (bi[:, None] == lax.broadcasted_iota(jnp.int32, (BR, 64), 1)
          ).astype(jnp.float32)
    ce = jnp.dot(oh, cf_ref[...], preferred_element_type=jnp.float32)
    v0, v1, v2 = vec[:, 0:1], vec[:, 1:2], vec[:, 2:3]
    er0 = ce[:, 0:1] * v0 + ce[:, 3:4] * v1 + ce[:, 6:7] * v2
    er1 = ce[:, 1:2] * v0 + ce[:, 4:5] * v1 + ce[:, 7:8] * v2
    er2 = ce[:, 2:3] * v0 + ce[:, 5:6] * v1 + ce[:, 8:9] * v2
    er_ref[...] = jnp.concatenate(
        [er0, er1, er2, jnp.zeros((BR, 125), jnp.float32)], axis=1)
    d = jnp.sqrt(er0 * er0 + er1 * er1 + er2 * er2)
    step = 8.0 / (EDGE_BINS - 1)
    centers = lax.broadcasted_iota(
        jnp.int32, (1, EDGE_BINS), 1).astype(jnp.float32) * step
    diff = d - centers
    y0_ref[...] = jnp.exp(-(1.0 / (step * step)) * diff * diff)


def _geo1(be3, ps, pd, tc16, cellflat):
    Rp = ps.shape[0]
    nb = Rp // BR
    v16 = pl.BlockSpec((BR, 128), lambda i: (i, 0))
    return pl.pallas_call(
        _geo1_body,
        grid=(nb,),
        in_specs=[pl.BlockSpec((1, 1, BR), lambda i: (i, 0, 0)),
                  v16, v16, v16,
                  pl.BlockSpec((64, 16), lambda i: (0, 0))],
        out_specs=[v16, pl.BlockSpec((BR, EDGE_BINS), lambda i: (i, 0))],
        out_shape=[jax.ShapeDtypeStruct((Rp, 128), jnp.float32),
                   jax.ShapeDtypeStruct((Rp, EDGE_BINS), jnp.float32)],
    )(be3, ps, pd, tc16, cellflat)


def _geo2_body(eu_ref, ev_ref, z0_ref):
    eu = eu_ref[...]
    ev = ev_ref[...]
    du = jnp.sum(eu * ev, axis=1, keepdims=True)
    nu = jnp.sum(eu * eu, axis=1, keepdims=True)
    nv = jnp.sum(ev * ev, axis=1, keepdims=True)
    cos = du / (jnp.sqrt(nu) * jnp.sqrt(nv))
    step = 2.0 / (TRI_BINS - 1)
    centers = -1.0 + lax.broadcasted_iota(
        jnp.int32, (1, TRI_BINS), 1).astype(jnp.float32) * step
    diff = cos - centers
    z0_ref[...] = jnp.exp(-(1.0 / (step * step)) * diff * diff)


def _geo2(eu, ev):
    Rp = eu.shape[0]
    nb = Rp // BR
    v16 = pl.BlockSpec((BR, 128), lambda i: (i, 0))
    return pl.pallas_call(
        _geo2_body,
        grid=(nb,),
        in_specs=[v16, v16],
        out_specs=pl.BlockSpec((BR, TRI_BINS), lambda i: (i, 0)),
        out_shape=jax.ShapeDtypeStruct((Rp, TRI_BINS), jnp.float32),
    )(eu, ev)


def _atom_body(z3_ref, zt_ref, w_ref, b_ref, o_ref, p_ref, *, rows):
    i = pl.program_id(0)
    zi = z3_ref[0, 0, :]
    oh = (zi[:, None] == lax.broadcasted_iota(jnp.int32, (BR, 100), 1)
          ).astype(jnp.float32)
    e = jnp.dot(oh, zt_ref[...], preferred_element_type=jnp.float32)
    z = jnp.dot(e, w_ref[...], preferred_element_type=jnp.float32) + b_ref[...]
    o_ref[...] = z
    s, q = _stats(z, i, rows)
    p_ref[0, 0, :] = s
    p_ref[0, 1, :] = q


def _atom_embed(z3, z_table, W, b, rows):
    Rp = z3.shape[0] * BR
    nb = z3.shape[0]
    return pl.pallas_call(
        functools.partial(_atom_body, rows=rows),
        grid=(nb,),
        in_specs=[pl.BlockSpec((1, 1, BR), lambda i: (i, 0, 0)),
                  pl.BlockSpec(z_table.shape, lambda i: (0, 0)),
                  pl.BlockSpec((z_table.shape[1], HID), lambda i: (0, 0)),
                  pl.BlockSpec((1, HID), lambda i: (0, 0))],
        out_specs=[pl.BlockSpec((BR, HID), lambda i: (i, 0)),
                   pl.BlockSpec((1, 8, HID), lambda i: (i, 0, 0))],
        out_shape=[jax.ShapeDtypeStruct((Rp, HID), jnp.float32),
                   jax.ShapeDtypeStruct((nb, 8, HID), jnp.float32)],
    )(z3, z_table, W, b.reshape(1, HID))


def _pool_body(x_ref, ba_ref, fw_ref, fb_ref, o_ref, h_ref, c_ref, *, rows,
               nseg, nb):
    i = pl.program_id(0)

    @pl.when(i == 0)
    def _():
        h_ref[...] = jnp.zeros_like(h_ref)
        c_ref[...] = jnp.zeros_like(c_ref)

    rid = i * BR + lax.broadcasted_iota(jnp.int32, (BR, 1), 0)
    bi = ba_ref[0, 0, :]
    oh = jnp.where(
        rid < rows,
        (bi[:, None] == lax.broadcasted_iota(jnp.int32, (BR, nseg), 1)
         ).astype(jnp.float32), 0.0)
    h_ref[...] += lax.dot_general(oh, x_ref[...], (((0,), (0,)), ((), ())),
                                  preferred_element_type=jnp.float32)
    c_ref[...] += jnp.sum(oh, axis=0)[:, None]

    @pl.when(i == nb - 1)
    def _():
        hm = h_ref[...] / jnp.maximum(c_ref[...], 1.0)
        o_ref[...] = jnp.dot(hm, fw_ref[...],
                             preferred_element_type=jnp.float32) + fb_ref[...]


def _pool(x, ba3, fc_W, fc_b, rows, nseg):
    Rp = x.shape[0]
    nb = Rp // BR
    return pl.pallas_call(
        functools.partial(_pool_body, rows=rows, nseg=nseg, nb=nb),
        grid=(nb,),
        in_specs=[pl.BlockSpec((BR, HID), lambda i: (i, 0)),
                  pl.BlockSpec((1, 1, BR), lambda i: (i, 0, 0)),
                  pl.BlockSpec((HID, 1), lambda i: (0, 0)),
                  pl.BlockSpec((1, 1), lambda i: (0, 0))],
        out_specs=pl.BlockSpec((nseg, 1), lambda i: (0, 0)),
        out_shape=jax.ShapeDtypeStruct((nseg, 1), jnp.float32),
        scratch_shapes=[pltpu.VMEM((nseg, HID), jnp.float32),
                        pltpu.VMEM((nseg, HID), jnp.float32)],
    )(x, ba3, fc_W, fc_b.reshape(1, 1))


def _sc_info():
    info = plsc.get_sparse_core_info()
    return info.num_cores, info.num_subcores


def _gather_rows(table, idx):
    V, D = table.shape
    B = idx.shape[0]
    nc, ns = _sc_info()
    nw = nc * ns
    bw = B // nw
    ch = 40960 // D
    nch = bw // ch
    mesh = plsc.VectorSubcoreMesh(core_axis_name="c", subcore_axis_name="s")

    @functools.partial(
        pl.kernel, mesh=mesh,
        out_type=jax.ShapeDtypeStruct((B, D), jnp.float32),
        scratch_types=[pltpu.VMEM((ch,), jnp.int32),
                       pltpu.VMEM((ch,), jnp.int32),
                       pltpu.VMEM((ch, D), jnp.float32),
                       pltpu.VMEM((ch, D), jnp.float32),
                       pltpu.SemaphoreType.DMA,
                       pltpu.SemaphoreType.DMA],
    )
    def k(table_hbm, idx_hbm, out_hbm, idx0, idx1, rows0, rows1, gs0, gs1):
        wid = lax.axis_index("s") * nc + lax.axis_index("c")
        base = wid * bw

        def drain(rows_v, gsem, off):
            pltpu.make_async_copy(
                out_hbm.at[pl.ds(0, ch)], rows_v, gsem).wait()
            pltpu.sync_copy(rows_v, out_hbm.at[pl.ds(off, ch)])

        def body(jj, carry):
            j0 = 2 * jj
            off0 = base + j0 * ch
            pltpu.sync_copy(idx_hbm.at[pl.ds(off0, ch)], idx0)
            pltpu.async_copy(table_hbm.at[idx0], rows0, gs0)

            @pl.when(jj > 0)
            def _():
                drain(rows1, gs1, off0 - ch)

            off1 = off0 + ch
            pltpu.sync_copy(idx_hbm.at[pl.ds(off1, ch)], idx1)
            pltpu.async_copy(table_hbm.at[idx1], rows1, gs1)
            drain(rows0, gs0, off0)
            return carry

        lax.fori_loop(0, nch // 2, body, 0)
        drain(rows1, gs1, base + (nch - 1) * ch)

    return k(table, idx)


def _segment_sum2(vals, idx, vpad):
    half = vals.shape[0] // 2
    s0 = jax.ops.segment_sum(vals[:half], idx[:half], num_segments=vpad)
    s1 = jax.ops.segment_sum(vals[half:], idx[half:], num_segments=vpad)
    return jnp.stack([s0, s1])


def _pad_rows(a, rp):
    pad = [(0, rp - a.shape[0])] + [(0, 0)] * (a.ndim - 1)
    return jnp.pad(a, pad)


def _eggc_pallas(srcp, dstp, dst_seg, nf, ef, W, b, bns, bnb, n_real, e_real,
                 vpad):
    Wcat = jnp.concatenate([W[0], W[4], W[1], W[3]], axis=1)
    bcat = jnp.concatenate([b[0], b[4], b[1], b[3]], axis=0)
    p1, p2, a3 = _nf_proj(nf, Wcat, bcat)
    g1 = _gather_rows(p1, srcp)
    g2 = _gather_rows(p2, dstp)
    m, sg, sh, pm = _msg(ef, g1, g2, W[2], b[2], e_real)
    s1 = _segment_sum2(sh, dst_seg, vpad)
    s2 = _segment_sum2(sg, dst_seg, vpad)
    xin, px = _gate(a3, s1, s2, n_real)
    x_new = _bn_silu(xin, px, bns[0], bnb[0], n_real, res=nf)
    y_new = _bn_silu(m, pm, bns[1], bnb[1], e_real, res=ef)
    return x_new, y_new


def kernel(edge_index, triplet_index, pos, cell, batch_edges, target_cell, z,
           batch_atoms, num_structures, z_table, atom_W, atom_b, atom_bn_s,
           atom_bn_b, edge_W1, edge_b1, edge_bn1_s, edge_bn1_b, edge_W2,
           edge_b2, edge_bn2_s, edge_bn2_b, ang_W1, ang_b1, ang_bn1_s,
           ang_bn1_b, ang_W2, ang_b2, ang_bn2_s, ang_bn2_b, eggc_W, eggc_b,
           eggc_bn_s, eggc_bn_b, fc_W, fc_b):
    n_nodes = pos.shape[0]
    n_edges = edge_index.shape[1]
    n_tri = triplet_index.shape[1]
    n_structures = cell.shape[0]
    npad = ((n_nodes + 2047) // 2048) * 2048
    epad = ((n_edges + 16383) // 16384) * 16384
    tpad = ((n_tri + 16383) // 16384) * 16384

    srcp = _pad_rows(edge_index[0].astype(jnp.int32), epad)
    dstp = _pad_rows(edge_index[1].astype(jnp.int32), epad)
    tup = _pad_rows(triplet_index[0].astype(jnp.int32), tpad)
    tvp = _pad_rows(triplet_index[1].astype(jnp.int32), tpad)
    dst_seg = jnp.where(
        jnp.arange(epad) < n_edges, dstp, jnp.int32(npad))
    tv_seg = jnp.where(
        jnp.arange(tpad) < n_tri, tvp, jnp.int32(epad))

    pos16 = _pad_rows(jnp.pad(pos, ((0, 0), (0, 125))), npad)
    tc16 = _pad_rows(jnp.pad(target_cell, ((0, 0), (0, 125))), epad)
    be3 = _pad_rows(batch_edges.astype(jnp.int32), epad).reshape(-1, 1, BR)
    ba3 = _pad_rows(batch_atoms.astype(jnp.int32), npad).reshape(-1, 1, BR)
    z3 = _pad_rows(z.astype(jnp.int32), npad).reshape(-1, 1, BR)
    cellflat = jnp.pad(cell.reshape(n_structures, 9), ((0, 0), (0, 7)))

    ps = _gather_rows(pos16, srcp)
    pd = _gather_rows(pos16, dstp)
    er16, y0 = _geo1(be3, ps, pd, tc16, cellflat)
    eu = _gather_rows(er16, tup)
    ev = _gather_rows(er16, tvp)
    z0 = _geo2(eu, ev)

    za, pa = _atom_embed(z3, z_table, atom_W, atom_b, n_nodes)
    x = _bn_silu(za, pa, atom_bn_s, atom_bn_b, n_nodes)
    z1, p1 = _linear_stats(y0, edge_W1, edge_b1, n_edges)
    y = _bn_silu(z1, p1, edge_bn1_s, edge_bn1_b, n_edges)
    z2, p2 = _linear_stats(y, edge_W2, edge_b2, n_edges)
    y = _bn_silu(z2, p2, edge_bn2_s, edge_bn2_b, n_edges)
    z3f, p3 = _linear_stats(z0, ang_W1, ang_b1, n_tri)
    zf = _bn_silu(z3f, p3, ang_bn1_s, ang_bn1_b, n_tri)
    z4, p4 = _linear_stats(zf, ang_W2, ang_b2, n_tri)
    zf = _bn_silu(z4, p4, ang_bn2_s, ang_bn2_b, n_tri)

    for i in range(N_ALIGNN):
        x, m = _eggc_pallas(srcp, dstp, dst_seg, x, y, eggc_W[2 * i],
                            eggc_b[2 * i], eggc_bn_s[2 * i], eggc_bn_b[2 * i],
                            n_nodes, n_edges, npad)
        y, zf = _eggc_pallas(tup, tvp, tv_seg, m, zf, eggc_W[2 * i + 1],
                             eggc_b[2 * i + 1], eggc_bn_s[2 * i + 1],
                             eggc_bn_b[2 * i + 1], n_edges, n_tri, epad)
    for i in range(N_GCN):
        k = 2 * N_ALIGNN + i
        x, y = _eggc_pallas(srcp, dstp, dst_seg, x, y, eggc_W[k], eggc_b[k],
                            eggc_bn_s[k], eggc_bn_b[k], n_nodes, n_edges,
                            npad)

    out = _pool(x, ba3, fc_W, fc_b, n_nodes, n_structures)
    return jnp.squeeze(out)
